# Initial kernel scaffold; baseline (speedup 1.0000x reference)
import functools

import jax
import jax.numpy as jnp
from jax import lax
from jax.experimental import pallas as pl
from jax.experimental.pallas import tpu as pltpu
from jax.experimental.pallas import tpu_sc as plsc

VOCAB = 1000000
EMBED_DIM = 64

_INFO = plsc.get_sparse_core_info()
_NC = _INFO.num_cores
_NS = _INFO.num_subcores
_NW = _NC * _NS

_CHUNK = 128


def _make_gather(n_chunks: int):
    mesh = plsc.VectorSubcoreMesh(core_axis_name="c", subcore_axis_name="s")

    @functools.partial(
        pl.kernel,
        out_type=jax.ShapeDtypeStruct((_NW, n_chunks, _CHUNK, EMBED_DIM), jnp.float32),
        mesh=mesh,
        scratch_types=[
            pltpu.VMEM((n_chunks, _CHUNK), jnp.int32),
            pltpu.VMEM((_CHUNK, EMBED_DIM), jnp.float32),
            pltpu.SemaphoreType.DMA,
        ],
    )
    def gather_kernel(idx_hbm, table_hbm, out_hbm, idx_v, rows_v, sem):
        wid = lax.axis_index("s") * _NC + lax.axis_index("c")
        pltpu.sync_copy(idx_hbm.at[wid], idx_v)

        def chunk_body(j, carry):
            pltpu.async_copy(table_hbm.at[idx_v.at[j]], rows_v, sem).wait()
            pltpu.sync_copy(rows_v, out_hbm.at[wid, j])
            return carry

        lax.fori_loop(0, n_chunks, chunk_body, 0)

    return gather_kernel


def kernel(tokens, weight):
    b, s = tokens.shape
    total = b * s
    assert total % (_NW * _CHUNK) == 0
    n_chunks = total // (_NW * _CHUNK)
    idx = tokens.reshape(_NW, n_chunks, _CHUNK).astype(jnp.int32)
    out = _make_gather(n_chunks)(idx, weight)
    return out.reshape(b, s, EMBED_DIM)


# speedup vs baseline: 1.6841x; 1.6841x over previous
import functools

import jax
import jax.numpy as jnp
from jax import lax
from jax.experimental import pallas as pl
from jax.experimental.pallas import tpu as pltpu
from jax.experimental.pallas import tpu_sc as plsc

VOCAB = 1000000
EMBED_DIM = 64

_INFO = plsc.get_sparse_core_info()
_NC = _INFO.num_cores
_NS = _INFO.num_subcores
_NW = _NC * _NS

_CHUNK = 128


def _make_gather(n_chunks: int):
    mesh = plsc.VectorSubcoreMesh(core_axis_name="c", subcore_axis_name="s")

    @functools.partial(
        pl.kernel,
        out_type=jax.ShapeDtypeStruct((_NW, n_chunks, _CHUNK, EMBED_DIM), jnp.float32),
        mesh=mesh,
        scratch_types=[
            pltpu.VMEM((n_chunks, _CHUNK), jnp.int32),
            pltpu.VMEM((_CHUNK, EMBED_DIM), jnp.float32),
            pltpu.SemaphoreType.DMA,
        ],
        compiler_params=pltpu.CompilerParams(use_tc_tiling_on_sc=False),
    )
    def gather_kernel(idx_hbm, table_hbm, out_hbm, idx_v, rows_v, sem):
        wid = lax.axis_index("s") * _NC + lax.axis_index("c")
        pltpu.sync_copy(idx_hbm.at[wid], idx_v)

        def chunk_body(j, carry):
            pltpu.async_copy(table_hbm.at[idx_v.at[j]], rows_v, sem).wait()
            pltpu.sync_copy(rows_v, out_hbm.at[wid, j])
            return carry

        lax.fori_loop(0, n_chunks, chunk_body, 0)

    return gather_kernel


def kernel(tokens, weight):
    b, s = tokens.shape
    total = b * s
    assert total % (_NW * _CHUNK) == 0
    n_chunks = total // (_NW * _CHUNK)
    idx = tokens.reshape(_NW, n_chunks, _CHUNK).astype(jnp.int32)
    out = _make_gather(n_chunks)(idx, weight)
    return out.reshape(b, s, EMBED_DIM)


# speedup vs baseline: 1.8720x; 1.1116x over previous
import functools

import jax
import jax.numpy as jnp
from jax import lax
from jax.experimental import pallas as pl
from jax.experimental.pallas import tpu as pltpu
from jax.experimental.pallas import tpu_sc as plsc

VOCAB = 1000000
EMBED_DIM = 64

try:
    _INFO = plsc.get_sparse_core_info()
    _NC = _INFO.num_cores
    _NS = _INFO.num_subcores
except Exception:
    _NC, _NS = 2, 16
_NW = _NC * _NS

_CHUNK = 128


_K = 4


def _make_gather(n_chunks: int):
    mesh = plsc.VectorSubcoreMesh(core_axis_name="c", subcore_axis_name="s")
    n_groups = n_chunks // _K
    assert n_chunks % _K == 0 and n_groups % 2 == 0

    @functools.partial(
        pl.kernel,
        out_type=jax.ShapeDtypeStruct((_NW, n_chunks, _CHUNK, EMBED_DIM), jnp.float32),
        mesh=mesh,
        scratch_types=[
            pltpu.VMEM((n_chunks, _CHUNK), jnp.int32),
            pltpu.VMEM((2, _K, _CHUNK, EMBED_DIM), jnp.float32),
            pltpu.SemaphoreType.DMA,
            pltpu.SemaphoreType.DMA,
            pltpu.SemaphoreType.DMA,
            pltpu.SemaphoreType.DMA,
        ],
        compiler_params=pltpu.CompilerParams(use_tc_tiling_on_sc=False),
    )
    def gather_kernel(idx_hbm, table_hbm, out_hbm, idx_v, rows_v, sem_ga, sem_gb,
                      sem_wa, sem_wb):
        wid = lax.axis_index("s") * _NC + lax.axis_index("c")
        pltpu.sync_copy(idx_hbm.at[wid], idx_v)

        def start_gathers(m, buf, sem):
            for b in range(_K):
                pltpu.async_copy(table_hbm.at[idx_v.at[m * _K + b]],
                                 rows_v.at[buf, b], sem)

        def wait_gathers(m, buf, sem):
            for b in range(_K):
                pltpu.make_async_copy(table_hbm.at[idx_v.at[m * _K + b]],
                                      rows_v.at[buf, b], sem).wait()

        def start_writes(m, buf, sem):
            pltpu.async_copy(rows_v.at[buf], out_hbm.at[wid, pl.ds(m * _K, _K)],
                             sem)

        def wait_writes(m, buf, sem):
            pltpu.make_async_copy(rows_v.at[buf],
                                  out_hbm.at[wid, pl.ds(m * _K, _K)], sem).wait()

        start_gathers(0, 0, sem_ga)

        def group_body(g, carry):
            m = 2 * g
            wait_gathers(m, 0, sem_ga)
            start_writes(m, 0, sem_wa)

            @pl.when(g > 0)
            def _():
                wait_writes(m - 1, 1, sem_wb)

            start_gathers(m + 1, 1, sem_gb)
            wait_gathers(m + 1, 1, sem_gb)
            start_writes(m + 1, 1, sem_wb)
            wait_writes(m, 0, sem_wa)

            @pl.when(m + 2 < n_groups)
            def _():
                start_gathers(m + 2, 0, sem_ga)

            return carry

        lax.fori_loop(0, n_groups // 2, group_body, 0)
        wait_writes(n_groups - 1, 1, sem_wb)

    return gather_kernel


def kernel(tokens, weight):
    b, s = tokens.shape
    total = b * s
    assert total % (_NW * _CHUNK) == 0
    n_chunks = total // (_NW * _CHUNK)
    idx = tokens.reshape(_NW, n_chunks, _CHUNK).astype(jnp.int32)
    out = _make_gather(n_chunks)(idx, weight)
    return out.reshape(b, s, EMBED_DIM)
